# Initial kernel scaffold; baseline (speedup 1.0000x reference)
import jax
import jax.numpy as jnp
from jax.experimental import pallas as pl


def kernel(input, weight):
    raise NotImplementedError("write your pallas kernel here")


# speedup vs baseline: 438.1698x; 438.1698x over previous
import functools

import jax
import jax.numpy as jnp
from jax.experimental import pallas as pl

M0 = 0.3
SIZE = (40, 40)
LIMS = [[1.0, -1.0], [-1.0, 1.0]]
N = SIZE[0] * SIZE[1]
ROWS = 400


def _grid_points():
    e0 = jnp.linspace(LIMS[0][0], LIMS[0][1], SIZE[0])
    e1 = jnp.linspace(LIMS[1][0], LIMS[1][1], SIZE[1])
    g = jnp.stack([jnp.tile(e1, SIZE[0]), jnp.repeat(e0, SIZE[1])], axis=1)
    return g.astype(jnp.float32)


def _dtm_body(y0_ref, y1_ref, xt_ref, w_ref, o_ref):
    y0 = y0_ref[:, :]
    y1 = y1_ref[:, :]
    x0 = xt_ref[0, 0:1, :]
    x1 = xt_ref[0, 1:2, :]
    w = w_ref[0, :, :]

    d0 = y0 - x0
    d1 = y1 - x1
    dist = jnp.sqrt(d0 * d0 + d1 * d1)
    t = dist * dist
    t_bits = jax.lax.bitcast_convert_type(t, jnp.int32)

    wb = M0 * jnp.sum(w)

    lo0 = jnp.full((ROWS, 1), -1, jnp.int32)
    hi0 = jnp.full((ROWS, 1), 0x7F800000, jnp.int32)

    def body(_, carry):
        lo, hi = carry
        mid = lo + ((hi - lo) >> 1)
        cnt = jnp.sum(jnp.where(t_bits <= mid, w, 0.0), axis=1,
                      keepdims=True)
        pred = cnt >= wb
        return jnp.where(pred, lo, mid), jnp.where(pred, mid, hi)

    lo, hi = jax.lax.fori_loop(0, 31, body, (lo0, hi0))

    t_star = jax.lax.bitcast_convert_type(hi, jnp.float32)
    mask = t_bits < hi
    s_w = jnp.sum(jnp.where(mask, w, 0.0), axis=1, keepdims=True)
    s_d = jnp.sum(jnp.where(mask, w * t, 0.0), axis=1, keepdims=True)
    val = s_d + t_star * (wb - s_w)
    o_ref[0, :, :] = jnp.sqrt(val / wb)


def kernel(input, weight):
    B = input.shape[0]
    g = _grid_points()
    y0 = g[:, 0:1]
    y1 = g[:, 1:2]
    xt = jnp.swapaxes(input, 1, 2)
    nb = N // ROWS
    out = pl.pallas_call(
        _dtm_body,
        grid=(B, nb),
        in_specs=[
            pl.BlockSpec((ROWS, 1), lambda b, rb: (rb, 0)),
            pl.BlockSpec((ROWS, 1), lambda b, rb: (rb, 0)),
            pl.BlockSpec((1, 2, N), lambda b, rb: (b, 0, 0)),
            pl.BlockSpec((1, 1, N), lambda b, rb: (b, 0, 0)),
        ],
        out_specs=pl.BlockSpec((1, ROWS, 1), lambda b, rb: (b, rb, 0)),
        out_shape=jax.ShapeDtypeStruct((B, N, 1), jnp.float32),
    )(y0, y1, xt, weight.reshape(B, 1, N))
    return out.reshape(B, N)
